# baseline (speedup 1.0000x reference)
import functools

import jax
import jax.numpy as jnp
from jax import lax
from jax.experimental import pallas as pl
from jax.experimental.pallas import tpu as pltpu
from jax.experimental.pallas import tpu_sc as plsc

GAMMA = 12.0
BATCH = 16384
EMB_DIM = 64

NUM_CORES = 2
NUM_SUBCORES = 16
NW = NUM_CORES * NUM_SUBCORES
ROWS_PER_W = BATCH // NW
CHUNK = 128
NCHUNK = ROWS_PER_W // CHUNK


def _sc_kernel(hidx_hbm, ridx_hbm, tidx_hbm, ent_hbm, rel_hbm, out_hbm,
               hidx_v, ridx_v, tidx_v, h_rows, r_rows, t_rows, out_v, sem):
    wid = lax.axis_index("s") * NUM_CORES + lax.axis_index("c")
    base = wid * ROWS_PER_W

    pltpu.sync_copy(hidx_hbm.at[wid], hidx_v)
    pltpu.sync_copy(ridx_hbm.at[wid], ridx_v)
    pltpu.sync_copy(tidx_hbm.at[wid], tidx_v)

    copies = []
    for c in range(NCHUNK):
        dst = pl.ds(c * CHUNK, CHUNK)
        copies.append(pltpu.async_copy(ent_hbm.at[hidx_v.at[c]], h_rows.at[dst, :], sem))
        copies.append(pltpu.async_copy(rel_hbm.at[ridx_v.at[c]], r_rows.at[dst, :], sem))
        copies.append(pltpu.async_copy(ent_hbm.at[tidx_v.at[c]], t_rows.at[dst, :], sem))
    for cp in copies:
        cp.wait()

    lanes = lax.iota(jnp.int32, 16)

    def body(g, _):
        scores = jnp.zeros((16,), jnp.float32)
        for i in range(16):
            b = g * 16 + i
            acc = jnp.zeros((16,), jnp.float32)
            for j in range(EMB_DIM // 16):
                sl = pl.ds(j * 16, 16)
                acc = acc + jnp.abs(h_rows[b, sl] + r_rows[b, sl] - t_rows[b, sl])
            tot = jnp.sum(acc)
            scores = jnp.where(lanes == i, tot, scores)
        out_v[pl.ds(g * 16, 16)] = GAMMA - scores
        return 0

    lax.fori_loop(0, ROWS_PER_W // 16, body, 0)

    pltpu.sync_copy(out_v, out_hbm.at[pl.ds(base, ROWS_PER_W)])


def kernel(sample, ent_emb, relation_embedding):
    sample_t = sample.T.astype(jnp.int32)
    hidx = sample_t[0].reshape(NW, NCHUNK, CHUNK)
    ridx = sample_t[1].reshape(NW, NCHUNK, CHUNK)
    tidx = sample_t[2].reshape(NW, NCHUNK, CHUNK)

    mesh = plsc.VectorSubcoreMesh(core_axis_name="c", subcore_axis_name="s")
    run = functools.partial(
        pl.kernel,
        mesh=mesh,
        compiler_params=pltpu.CompilerParams(
            needs_layout_passes=False, use_tc_tiling_on_sc=False),
        out_type=jax.ShapeDtypeStruct((BATCH,), jnp.float32),
        scratch_types=[
            pltpu.VMEM((NCHUNK, CHUNK), jnp.int32),
            pltpu.VMEM((NCHUNK, CHUNK), jnp.int32),
            pltpu.VMEM((NCHUNK, CHUNK), jnp.int32),
            pltpu.VMEM((ROWS_PER_W, EMB_DIM), jnp.float32),
            pltpu.VMEM((ROWS_PER_W, EMB_DIM), jnp.float32),
            pltpu.VMEM((ROWS_PER_W, EMB_DIM), jnp.float32),
            pltpu.VMEM((ROWS_PER_W,), jnp.float32),
            pltpu.SemaphoreType.DMA,
        ],
    )(_sc_kernel)
    score = run(hidx, ridx, tidx, ent_emb, relation_embedding)
    return score.reshape(BATCH, 1)
